# baseline (speedup 1.0000x reference)
import functools

import jax
import jax.numpy as jnp
import numpy as np
from jax import lax
from jax.experimental import pallas as pl
from jax.experimental.pallas import tpu as pltpu
from jax.experimental.pallas import tpu_sc as plsc

_ROWS = 128
_LANES = 16


def _sc_gather_fn(B, DEG, F, NC, NS):
    NW = NC * NS
    BPW = B // NW
    NPC = _ROWS // DEG
    NCHUNK = (BPW * DEG) // _ROWS
    NF = F // _LANES
    FP = F // 2
    SELF_CHUNKS = BPW // _ROWS

    mesh = plsc.VectorSubcoreMesh(core_axis_name="c", subcore_axis_name="s")

    @functools.partial(
        pl.kernel,
        out_type=(jax.ShapeDtypeStruct((B, FP), jnp.uint32),
                  jax.ShapeDtypeStruct((B, FP), jnp.uint32)),
        mesh=mesh,
        compiler_params=pltpu.CompilerParams(needs_layout_passes=False),
        scratch_types=[
            pltpu.VMEM((BPW,), jnp.int32),
            pltpu.VMEM((NCHUNK, _ROWS), jnp.int32),
            pltpu.VMEM((_ROWS, F), jnp.float32),
            pltpu.VMEM((_ROWS, F), jnp.float32),
            pltpu.VMEM((_ROWS, F), jnp.float32),
            pltpu.VMEM((_ROWS, F), jnp.float32),
            pltpu.VMEM((_ROWS, FP), jnp.uint32),
            pltpu.VMEM((BPW, FP), jnp.uint32),
            pltpu.SemaphoreType.DMA,
            pltpu.SemaphoreType.DMA,
            pltpu.SemaphoreType.DMA,
            pltpu.SemaphoreType.DMA,
        ],
    )
    def k(feat_hbm, ni_hbm, bn_hbm, self_hbm, nsum_hbm, bn_v, ni_v,
          buf0, buf1, buf2, buf3, stage_v, acc_v, sem0, sem1, sem2, sem3):
        wid = lax.axis_index("s") * NC + lax.axis_index("c")
        base = wid * BPW
        bufs = (buf0, buf1, buf2, buf3)
        sems = (sem0, sem1, sem2, sem3)

        def pack_row(src, r, dst, dr):
            for f in range(NF // 2):
                a = src[r, pl.ds(32 * f, _LANES)]
                b = src[r, pl.ds(32 * f + _LANES, _LANES)]
                pk = plsc.pack(a, b, format=plsc.PackFormat.INTERLEAVED)
                dst[dr, pl.ds(_LANES * f, _LANES)] = plsc.bitcast(
                    pk, jnp.uint32)

        pltpu.sync_copy(bn_hbm.at[pl.ds(base, BPW)], bn_v)
        pltpu.sync_copy(ni_hbm.at[pl.ds(wid * NCHUNK, NCHUNK)], ni_v)

        for c in range(SELF_CHUNKS):
            pltpu.async_copy(feat_hbm.at[bn_v.at[pl.ds(c * _ROWS, _ROWS)]],
                             bufs[c], sems[c])
        for n in range(2):
            pltpu.async_copy(feat_hbm.at[ni_v.at[n]], bufs[2 + n], sems[2 + n])

        for c in range(SELF_CHUNKS):
            pltpu.make_async_copy(feat_hbm.at[bn_v.at[pl.ds(c * _ROWS, _ROWS)]],
                                  bufs[c], sems[c]).wait()

            @pl.loop(0, _ROWS, unroll=4)
            def _(r, c=c):
                pack_row(bufs[c], r, stage_v, r)

            pltpu.sync_copy(stage_v,
                            self_hbm.at[pl.ds(base + c * _ROWS, _ROWS)])
        for n in range(2, 4):
            pltpu.async_copy(feat_hbm.at[ni_v.at[n]], bufs[n - 2],
                             sems[n - 2])

        @pl.loop(0, NCHUNK, step=4)
        def _(g):
            for b in range(4):
                chunk = g + b
                buf = bufs[(b + 2) % 4]
                sem = sems[(b + 2) % 4]
                pltpu.make_async_copy(feat_hbm.at[ni_v.at[chunk]], buf,
                                      sem).wait()
                for j in range(NPC):
                    rb = j * DEG

                    @pl.loop(
                        0, DEG,
                        init_carry=tuple(
                            jnp.zeros((_LANES,), jnp.float32)
                            for _ in range(NF)),
                        unroll=8)
                    def accs(r, carry, rb=rb, buf=buf):
                        return tuple(
                            carry[f] + buf[rb + r, pl.ds(f * _LANES, _LANES)]
                            for f in range(NF))

                    node = chunk * NPC + j
                    for f in range(NF // 2):
                        pk = plsc.pack(
                            accs[2 * f], accs[2 * f + 1],
                            format=plsc.PackFormat.INTERLEAVED)
                        acc_v[node, pl.ds(_LANES * f, _LANES)] = plsc.bitcast(
                            pk, jnp.uint32)

                @pl.when(chunk + 4 < NCHUNK)
                def _(buf=buf, sem=sem, chunk=chunk):
                    pltpu.async_copy(feat_hbm.at[ni_v.at[chunk + 4]], buf, sem)

        pltpu.sync_copy(acc_v, nsum_hbm.at[pl.ds(base, BPW)])

    return k


def _tc_head_fn(B, DEG, F, H, C, BLK):
    inv_deg = 1.0 / DEG
    FP = F // 2

    def unpack_lo(p):
        return lax.bitcast_convert_type(p << 16, jnp.float32)

    def unpack_hi(p):
        return lax.bitcast_convert_type(p & jnp.uint32(0xFFFF0000),
                                        jnp.float32)

    def body(s_ref, n_ref, w1l_ref, w1h_ref, w2l_ref, w2h_ref, wc_ref, o_ref):
        s = s_ref[...]
        n = n_ref[...]
        x = jnp.dot(unpack_lo(s), w1l_ref[...],
                    preferred_element_type=jnp.float32)
        x = x + jnp.dot(unpack_hi(s), w1h_ref[...],
                        preferred_element_type=jnp.float32)
        x = x + jnp.dot(unpack_lo(n), w2l_ref[...] * inv_deg,
                        preferred_element_type=jnp.float32)
        x = x + jnp.dot(unpack_hi(n), w2h_ref[...] * inv_deg,
                        preferred_element_type=jnp.float32)
        h = jnp.maximum(x, 0.0)
        o_ref[...] = jnp.dot(h, wc_ref[...], preferred_element_type=jnp.float32)

    return pl.pallas_call(
        body,
        grid=(B // BLK,),
        in_specs=[
            pl.BlockSpec((BLK, FP), lambda i: (i, 0)),
            pl.BlockSpec((BLK, FP), lambda i: (i, 0)),
            pl.BlockSpec((FP, H), lambda i: (0, 0)),
            pl.BlockSpec((FP, H), lambda i: (0, 0)),
            pl.BlockSpec((FP, H), lambda i: (0, 0)),
            pl.BlockSpec((FP, H), lambda i: (0, 0)),
            pl.BlockSpec((H, C), lambda i: (0, 0)),
        ],
        out_specs=pl.BlockSpec((BLK, C), lambda i: (i, 0)),
        out_shape=jax.ShapeDtypeStruct((B, C), jnp.float32),
    )


def kernel(features, neigh_idx, batch_nodes, W_enc, W_cls):
    B, DEG = neigh_idx.shape
    N, F = features.shape
    H = W_enc.shape[1]
    C = W_cls.shape[1]

    info = plsc.get_sparse_core_info()
    NC, NS = info.num_cores, info.num_subcores

    ni = neigh_idx.astype(jnp.int32).reshape(B * DEG // _ROWS, _ROWS)
    bn = batch_nodes.astype(jnp.int32)

    self_pk, nsum_pk = _sc_gather_fn(B, DEG, F, NC, NS)(features, ni, bn)

    perm = np.arange(F).reshape(F // 32, 2, _LANES)
    lo = perm[:, 0, :].reshape(-1)
    hi = perm[:, 1, :].reshape(-1)
    W1, W2 = W_enc[:F], W_enc[F:]
    scores = _tc_head_fn(B, DEG, F, H, C, BLK=512)(
        self_pk, nsum_pk, W1[lo], W1[hi], W2[lo], W2[hi], W_cls)
    return scores


# speedup vs baseline: 1.1264x; 1.1264x over previous
import functools

import jax
import jax.numpy as jnp
from jax import lax
from jax.experimental import pallas as pl
from jax.experimental.pallas import tpu as pltpu
from jax.experimental.pallas import tpu_sc as plsc

_ROWS = 128
_LANES = 16


def _sc_gather_fn(B, DEG, F, NC, NS):
    NW = NC * NS
    BPW = B // NW
    NPC = _ROWS // DEG
    NCHUNK = (BPW * DEG) // _ROWS
    NF = F // _LANES
    SELF_CHUNKS = BPW // _ROWS

    mesh = plsc.VectorSubcoreMesh(core_axis_name="c", subcore_axis_name="s")

    @functools.partial(
        pl.kernel,
        out_type=(jax.ShapeDtypeStruct((B, F), jnp.float32),
                  jax.ShapeDtypeStruct((B, F), jnp.float32)),
        mesh=mesh,
        scratch_types=[
            pltpu.VMEM((SELF_CHUNKS, _ROWS), jnp.int32),
            pltpu.VMEM((NCHUNK, _ROWS), jnp.int32),
            pltpu.VMEM((_ROWS, F), jnp.float32),
            pltpu.VMEM((_ROWS, F), jnp.float32),
            pltpu.VMEM((_ROWS, F), jnp.float32),
            pltpu.VMEM((_ROWS, F), jnp.float32),
            pltpu.VMEM((BPW, F), jnp.float32),
            pltpu.SemaphoreType.DMA,
            pltpu.SemaphoreType.DMA,
            pltpu.SemaphoreType.DMA,
            pltpu.SemaphoreType.DMA,
        ],
    )
    def k(feat_hbm, ni_hbm, bn_hbm, self_hbm, nsum_hbm, bn_v, ni_v,
          buf0, buf1, buf2, buf3, acc_v, sem0, sem1, sem2, sem3):
        wid = lax.axis_index("s") * NC + lax.axis_index("c")
        base = wid * BPW
        bufs = (buf0, buf1, buf2, buf3)
        sems = (sem0, sem1, sem2, sem3)

        pltpu.sync_copy(bn_hbm.at[pl.ds(wid * SELF_CHUNKS, SELF_CHUNKS)], bn_v)
        pltpu.sync_copy(ni_hbm.at[pl.ds(wid * NCHUNK, NCHUNK)], ni_v)

        for c in range(SELF_CHUNKS):
            pltpu.async_copy(feat_hbm.at[bn_v.at[c]], bufs[c], sems[c])
        for n in range(2):
            pltpu.async_copy(feat_hbm.at[ni_v.at[n]], bufs[2 + n], sems[2 + n])

        for c in range(SELF_CHUNKS):
            pltpu.make_async_copy(feat_hbm.at[bn_v.at[c]], bufs[c],
                                  sems[c]).wait()
            pltpu.sync_copy(bufs[c], self_hbm.at[pl.ds(base + c * _ROWS,
                                                       _ROWS)])
        for n in range(2, 4):
            pltpu.async_copy(feat_hbm.at[ni_v.at[n]], bufs[n - 2],
                             sems[n - 2])

        @pl.loop(0, NCHUNK, step=4)
        def _(g):
            for b in range(4):
                chunk = g + b
                buf = bufs[(b + 2) % 4]
                sem = sems[(b + 2) % 4]
                pltpu.make_async_copy(feat_hbm.at[ni_v.at[chunk]], buf,
                                      sem).wait()
                for j in range(NPC):
                    rb = j * DEG

                    @pl.loop(
                        0, DEG,
                        init_carry=tuple(
                            jnp.zeros((_LANES,), jnp.float32)
                            for _ in range(NF)),
                        unroll=8)
                    def accs(r, carry, rb=rb, buf=buf):
                        return tuple(
                            carry[f] + buf[rb + r, pl.ds(f * _LANES, _LANES)]
                            for f in range(NF))

                    node = chunk * NPC + j
                    for f in range(NF):
                        acc_v[node, pl.ds(f * _LANES, _LANES)] = accs[f]

                @pl.when(chunk + 4 < NCHUNK)
                def _(buf=buf, sem=sem, chunk=chunk):
                    pltpu.async_copy(feat_hbm.at[ni_v.at[chunk + 4]], buf, sem)

        pltpu.sync_copy(acc_v, nsum_hbm.at[pl.ds(base, BPW)])

    return k


def _tc_head_fn(B, DEG, F, H, C, BLK):
    inv_deg = 1.0 / DEG

    def body(s_ref, n_ref, w1_ref, w2_ref, wc_ref, o_ref):
        x = jnp.dot(s_ref[...], w1_ref[...],
                    preferred_element_type=jnp.float32)
        x = x + jnp.dot(n_ref[...] * inv_deg, w2_ref[...],
                        preferred_element_type=jnp.float32)
        h = jnp.maximum(x, 0.0)
        o_ref[...] = jnp.dot(h, wc_ref[...], preferred_element_type=jnp.float32)

    return pl.pallas_call(
        body,
        grid=(B // BLK,),
        in_specs=[
            pl.BlockSpec((BLK, F), lambda i: (i, 0)),
            pl.BlockSpec((BLK, F), lambda i: (i, 0)),
            pl.BlockSpec((F, H), lambda i: (0, 0)),
            pl.BlockSpec((F, H), lambda i: (0, 0)),
            pl.BlockSpec((H, C), lambda i: (0, 0)),
        ],
        out_specs=pl.BlockSpec((BLK, C), lambda i: (i, 0)),
        out_shape=jax.ShapeDtypeStruct((B, C), jnp.float32),
        compiler_params=pltpu.CompilerParams(
            dimension_semantics=("arbitrary",)),
    )


def kernel(features, neigh_idx, batch_nodes, W_enc, W_cls):
    B, DEG = neigh_idx.shape
    N, F = features.shape
    H = W_enc.shape[1]
    C = W_cls.shape[1]

    info = plsc.get_sparse_core_info()
    NC, NS = info.num_cores, info.num_subcores

    ni = neigh_idx.astype(jnp.int32).reshape(B * DEG // _ROWS, _ROWS)
    bn = batch_nodes.astype(jnp.int32).reshape(B // _ROWS, _ROWS)

    self32, nsum32 = _sc_gather_fn(B, DEG, F, NC, NS)(features, ni, bn)
    scores = _tc_head_fn(B, DEG, F, H, C, BLK=2048)(
        self32, nsum32, W_enc[:F], W_enc[F:], W_cls)
    return scores


# speedup vs baseline: 1.1282x; 1.0016x over previous
import functools

import jax
import jax.numpy as jnp
from jax import lax
from jax.experimental import pallas as pl
from jax.experimental.pallas import tpu as pltpu
from jax.experimental.pallas import tpu_sc as plsc

_ROWS = 128
_LANES = 16


def _sc_gather_fn(B, DEG, F, NC, NS):
    NW = NC * NS
    BPW = B // NW
    NPC = _ROWS // DEG
    NCHUNK = (BPW * DEG) // _ROWS
    NF = F // _LANES
    SELF_CHUNKS = BPW // _ROWS

    mesh = plsc.VectorSubcoreMesh(core_axis_name="c", subcore_axis_name="s")

    @functools.partial(
        pl.kernel,
        out_type=(jax.ShapeDtypeStruct((B, F), jnp.float32),
                  jax.ShapeDtypeStruct((B, F), jnp.float32)),
        mesh=mesh,
        scratch_types=[
            pltpu.VMEM((BPW,), jnp.int32),
            pltpu.VMEM((BPW, DEG), jnp.int32),
            pltpu.VMEM((NCHUNK, _ROWS), jnp.int32),
            pltpu.VMEM((_ROWS, F), jnp.float32),
            pltpu.VMEM((_ROWS, F), jnp.float32),
            pltpu.VMEM((_ROWS, F), jnp.float32),
            pltpu.VMEM((BPW, F), jnp.float32),
            pltpu.SemaphoreType.DMA,
            pltpu.SemaphoreType.DMA,
            pltpu.SemaphoreType.DMA,
        ],
    )
    def k(feat_hbm, ni_hbm, bn_hbm, self_hbm, nsum_hbm, bn_v, slab_v, ni_v,
          buf0, buf1, buf2, acc_v, sem0, sem1, sem2):
        wid = lax.axis_index("s") * NC + lax.axis_index("c")
        base = wid * BPW
        bufs = (buf0, buf1, buf2)
        sems = (sem0, sem1, sem2)

        pltpu.sync_copy(bn_hbm.at[pl.ds(base, BPW)], bn_v)
        pltpu.sync_copy(ni_hbm.at[pl.ds(base, BPW)], slab_v)

        for c in range(SELF_CHUNKS):
            pltpu.async_copy(feat_hbm.at[bn_v.at[pl.ds(c * _ROWS, _ROWS)]],
                             bufs[c], sems[c])

        @pl.loop(0, NCHUNK)
        def _(kk):
            for j in range(NPC):
                for c2 in range(DEG // _LANES):
                    ni_v[kk, pl.ds(j * DEG + c2 * _LANES, _LANES)] = (
                        slab_v[kk * NPC + j, pl.ds(c2 * _LANES, _LANES)])

        for c in range(SELF_CHUNKS):
            pltpu.make_async_copy(feat_hbm.at[bn_v.at[pl.ds(c * _ROWS, _ROWS)]],
                                  bufs[c], sems[c]).wait()
            pltpu.sync_copy(bufs[c], self_hbm.at[pl.ds(base + c * _ROWS,
                                                       _ROWS)])
            pltpu.async_copy(feat_hbm.at[ni_v.at[c]], bufs[c], sems[c])
        pltpu.async_copy(feat_hbm.at[ni_v.at[2]], bufs[2], sems[2])

        @pl.loop(0, NCHUNK + 2, step=3)
        def _(g):
            for b in range(3):
                chunk = g + b

                @pl.when(chunk < NCHUNK)
                def _(chunk=chunk, b=b):
                    buf = bufs[b]
                    sem = sems[b]
                    pltpu.make_async_copy(feat_hbm.at[ni_v.at[chunk]], buf,
                                          sem).wait()
                    for j in range(NPC):
                        rb = j * DEG

                        @pl.loop(
                            0, DEG,
                            init_carry=tuple(
                                jnp.zeros((_LANES,), jnp.float32)
                                for _ in range(NF)),
                            unroll=8)
                        def accs(r, carry, rb=rb, buf=buf):
                            return tuple(
                                carry[f] +
                                buf[rb + r, pl.ds(f * _LANES, _LANES)]
                                for f in range(NF))

                        node = chunk * NPC + j
                        for f in range(NF):
                            acc_v[node, pl.ds(f * _LANES, _LANES)] = accs[f]

                    @pl.when(chunk + 3 < NCHUNK)
                    def _(buf=buf, sem=sem, chunk=chunk):
                        pltpu.async_copy(feat_hbm.at[ni_v.at[chunk + 3]], buf,
                                         sem)

        pltpu.sync_copy(acc_v, nsum_hbm.at[pl.ds(base, BPW)])

    return k


def _tc_head_fn(B, DEG, F, H, C, BLK):
    inv_deg = 1.0 / DEG

    def body(s_ref, n_ref, w1_ref, w2_ref, wc_ref, o_ref):
        x = jnp.dot(s_ref[...], w1_ref[...],
                    preferred_element_type=jnp.float32)
        x = x + jnp.dot(n_ref[...] * inv_deg, w2_ref[...],
                        preferred_element_type=jnp.float32)
        h = jnp.maximum(x, 0.0)
        o_ref[...] = jnp.dot(h, wc_ref[...], preferred_element_type=jnp.float32)

    return pl.pallas_call(
        body,
        grid=(B // BLK,),
        in_specs=[
            pl.BlockSpec((BLK, F), lambda i: (i, 0)),
            pl.BlockSpec((BLK, F), lambda i: (i, 0)),
            pl.BlockSpec((F, H), lambda i: (0, 0)),
            pl.BlockSpec((F, H), lambda i: (0, 0)),
            pl.BlockSpec((H, C), lambda i: (0, 0)),
        ],
        out_specs=pl.BlockSpec((BLK, C), lambda i: (i, 0)),
        out_shape=jax.ShapeDtypeStruct((B, C), jnp.float32),
        compiler_params=pltpu.CompilerParams(
            dimension_semantics=("arbitrary",)),
    )


def kernel(features, neigh_idx, batch_nodes, W_enc, W_cls):
    B, DEG = neigh_idx.shape
    N, F = features.shape
    H = W_enc.shape[1]
    C = W_cls.shape[1]

    info = plsc.get_sparse_core_info()
    NC, NS = info.num_cores, info.num_subcores

    ni = neigh_idx.astype(jnp.int32)
    bn = batch_nodes.astype(jnp.int32)

    self32, nsum32 = _sc_gather_fn(B, DEG, F, NC, NS)(features, ni, bn)
    scores = _tc_head_fn(B, DEG, F, H, C, BLK=2048)(
        self32, nsum32, W_enc[:F], W_enc[F:], W_cls)
    return scores


# speedup vs baseline: 1.1507x; 1.0199x over previous
import functools

import jax
import jax.numpy as jnp
from jax import lax
from jax.experimental import pallas as pl
from jax.experimental.pallas import tpu as pltpu
from jax.experimental.pallas import tpu_sc as plsc

_ROWS = 128
_LANES = 16


def _sc_gather_fn(B, DEG, F, NC, NS):
    NW = NC * NS
    BPW = B // NW
    NPC = _ROWS // DEG
    NCHUNK = (BPW * DEG) // _ROWS
    NF = F // _LANES
    SELF_CHUNKS = BPW // _ROWS

    mesh = plsc.VectorSubcoreMesh(core_axis_name="c", subcore_axis_name="s")

    @functools.partial(
        pl.kernel,
        out_type=(jax.ShapeDtypeStruct((B, F), jnp.float32),
                  jax.ShapeDtypeStruct((B, F), jnp.float32)),
        mesh=mesh,
        scratch_types=[
            pltpu.VMEM((BPW,), jnp.int32),
            pltpu.VMEM((BPW, DEG), jnp.int32),
            pltpu.VMEM((NCHUNK, _ROWS), jnp.int32),
            pltpu.VMEM((_ROWS, F), jnp.float32),
            pltpu.VMEM((_ROWS, F), jnp.float32),
            pltpu.VMEM((_ROWS, F), jnp.float32),
            pltpu.VMEM((BPW, F), jnp.float32),
            pltpu.SemaphoreType.DMA,
            pltpu.SemaphoreType.DMA,
            pltpu.SemaphoreType.DMA,
        ],
    )
    def k(feat_hbm, ni_hbm, bn_hbm, self_hbm, nsum_hbm, bn_v, slab_v, ni_v,
          buf0, buf1, buf2, acc_v, sem0, sem1, sem2):
        wid = lax.axis_index("s") * NC + lax.axis_index("c")
        base = wid * BPW
        bufs = (buf0, buf1, buf2)
        sems = (sem0, sem1, sem2)

        pltpu.sync_copy(bn_hbm.at[pl.ds(base, BPW)], bn_v)
        pltpu.sync_copy(ni_hbm.at[pl.ds(base, BPW)], slab_v)

        for c in range(SELF_CHUNKS):
            pltpu.async_copy(feat_hbm.at[bn_v.at[pl.ds(c * _ROWS, _ROWS)]],
                             bufs[c], sems[c])

        @pl.loop(0, NCHUNK)
        def _(kk):
            for j in range(NPC):
                for c2 in range(DEG // _LANES):
                    ni_v[kk, pl.ds(j * DEG + c2 * _LANES, _LANES)] = (
                        slab_v[kk * NPC + j, pl.ds(c2 * _LANES, _LANES)])

        for c in range(SELF_CHUNKS):
            pltpu.make_async_copy(feat_hbm.at[bn_v.at[pl.ds(c * _ROWS, _ROWS)]],
                                  bufs[c], sems[c]).wait()
            pltpu.sync_copy(bufs[c], self_hbm.at[pl.ds(base + c * _ROWS,
                                                       _ROWS)])
            pltpu.async_copy(feat_hbm.at[ni_v.at[c]], bufs[c], sems[c])
        pltpu.async_copy(feat_hbm.at[ni_v.at[2]], bufs[2], sems[2])

        @pl.loop(0, NCHUNK + 2, step=3)
        def _(g):
            for b in range(3):
                chunk = g + b

                @pl.when(chunk < NCHUNK)
                def _(chunk=chunk, b=b):
                    buf = bufs[b]
                    sem = sems[b]
                    pltpu.make_async_copy(feat_hbm.at[ni_v.at[chunk]], buf,
                                          sem).wait()
                    for j in range(NPC):
                        rb = j * DEG

                        @pl.loop(
                            0, DEG,
                            init_carry=tuple(
                                jnp.zeros((_LANES,), jnp.float32)
                                for _ in range(NF)),
                            unroll=8)
                        def accs(r, carry, rb=rb, buf=buf):
                            return tuple(
                                carry[f] +
                                buf[rb + r, pl.ds(f * _LANES, _LANES)]
                                for f in range(NF))

                        node = chunk * NPC + j
                        for f in range(NF):
                            acc_v[node, pl.ds(f * _LANES, _LANES)] = accs[f]

                    @pl.when(chunk + 3 < NCHUNK)
                    def _(buf=buf, sem=sem, chunk=chunk):
                        pltpu.async_copy(feat_hbm.at[ni_v.at[chunk + 3]], buf,
                                         sem)

        pltpu.sync_copy(acc_v, nsum_hbm.at[pl.ds(base, BPW)])

    return k


def _tc_head_fn(B, DEG, F, H, C, BLK):
    inv_deg = 1.0 / DEG

    def body(s_ref, n_ref, w1_ref, w2_ref, wc_ref, o_ref):
        x = jnp.dot(s_ref[...], w1_ref[...],
                    preferred_element_type=jnp.float32)
        x = x + jnp.dot(n_ref[...] * inv_deg, w2_ref[...],
                        preferred_element_type=jnp.float32)
        h = jnp.maximum(x, 0.0)
        o_ref[...] = jnp.dot(h, wc_ref[...], preferred_element_type=jnp.float32)

    return pl.pallas_call(
        body,
        grid=(B // BLK,),
        in_specs=[
            pl.BlockSpec((BLK, F), lambda i: (i, 0)),
            pl.BlockSpec((BLK, F), lambda i: (i, 0)),
            pl.BlockSpec((F, H), lambda i: (0, 0)),
            pl.BlockSpec((F, H), lambda i: (0, 0)),
            pl.BlockSpec((H, C), lambda i: (0, 0)),
        ],
        out_specs=pl.BlockSpec((BLK, C), lambda i: (i, 0)),
        out_shape=jax.ShapeDtypeStruct((B, C), jnp.float32),
        compiler_params=pltpu.CompilerParams(
            dimension_semantics=("arbitrary",)),
    )


def kernel(features, neigh_idx, batch_nodes, W_enc, W_cls):
    B, DEG = neigh_idx.shape
    N, F = features.shape
    H = W_enc.shape[1]
    C = W_cls.shape[1]

    info = plsc.get_sparse_core_info()
    NC, NS = info.num_cores, info.num_subcores

    ni = neigh_idx.astype(jnp.int32)
    bn = batch_nodes.astype(jnp.int32)

    self32, nsum32 = _sc_gather_fn(B, DEG, F, NC, NS)(features, ni, bn)
    scores = _tc_head_fn(B, DEG, F, H, C, BLK=4096)(
        self32, nsum32, W_enc[:F], W_enc[F:], W_cls)
    return scores
